# baseline (speedup 1.0000x reference)
import functools
import jax
import jax.numpy as jnp
from jax import lax
from jax.experimental import pallas as pl
from jax.experimental.pallas import tpu as pltpu
from jax.experimental.pallas import tpu_sc as plsc

_LOSS_WEIGHT = 1.0

_N, _C, _R = 4, 96, 1152
_PLANE = _R * 128
_NW = 32
_HSL = _PLANE // _NW
_C_SC = 24
_C_TC = _C - _C_SC
_PB = 4


def _tc_body(p_ref, t_ref, m_ref, out_ref):
    n = pl.program_id(0)
    j = pl.program_id(1)
    nn = pl.num_programs(0)
    nj = pl.num_programs(1)

    @pl.when((n == 0) & (j == 0))
    def _init():
        out_ref[0] = 0.0
        out_ref[1] = 0.0
        out_ref[2] = 0.0

    m = m_ref[0]
    p = p_ref[...].reshape(_PB, _R, 128)
    t = t_ref[...].reshape(_PB, _R, 128)
    ad = jnp.abs(p - t)
    mb = m[None, :, :]
    adm = ad * mb
    cm = jnp.minimum(adm, mb)
    sm = cm * (adm - 0.5 * cm)
    out_ref[0] += jnp.sum(sm)
    out_ref[1] += jnp.sum(adm)

    @pl.when(j == 0)
    def _cnt():
        out_ref[2] += jnp.sum(m)


def _tc_partials(p3, t3, m3):
    nj = _C_TC // _PB
    return pl.pallas_call(
        _tc_body,
        grid=(_N, nj),
        in_specs=[
            pl.BlockSpec((_PB, _R, 128), lambda n, j: (n * (_C // _PB) + j, 0, 0)),
            pl.BlockSpec((_PB, _R, 128), lambda n, j: (n * (_C // _PB) + j, 0, 0)),
            pl.BlockSpec((1, _R, 128), lambda n, j: (n, 0, 0)),
        ],
        out_specs=pl.BlockSpec(memory_space=pltpu.SMEM),
        out_shape=jax.ShapeDtypeStruct((3,), jnp.float32),
    )(p3, t3, m3)


def _sc_body(p_hbm, t_hbm, m_hbm, out_hbm,
             pb0, pb1, tb0, tb1, mb, accb,
             ps0, ps1, ts0, ts1):
    wid = lax.axis_index("s") * 2 + lax.axis_index("c")
    hw0 = wid * _HSL

    for n in range(_N):
        pltpu.sync_copy(m_hbm.at[n, pl.ds(hw0, _HSL)], mb.at[n])

    pbufs = (pb0, pb1)
    tbufs = (tb0, tb1)
    psems = (ps0, ps1)
    tsems = (ts0, ts1)

    nq = _N * _C_SC

    def plane_off(q):
        n = q // _C_SC
        c = _C_TC + q % _C_SC
        return (n * _C + c) * _PLANE + hw0

    def issue(q, b):
        off = plane_off(q)
        pltpu.async_copy(p_hbm.at[pl.ds(off, _HSL)], pbufs[b], psems[b])
        pltpu.async_copy(t_hbm.at[pl.ds(off, _HSL)], tbufs[b], tsems[b])

    issue(0, 0)
    issue(1, 1)

    def chunk(q, b, acc):
        n = q // _C_SC
        off = plane_off(q)
        pltpu.make_async_copy(p_hbm.at[pl.ds(off, _HSL)], pbufs[b], psems[b]).wait()
        pltpu.make_async_copy(t_hbm.at[pl.ds(off, _HSL)], tbufs[b], tsems[b]).wait()

        @pl.when(q + 2 < nq)
        def _():
            issue(q + 2, b)

        def inner(i2, acc):
            acc_sm, acc_ad = acc
            for u in range(4):
                i = (i2 * 4 + u) * 16
                p = pbufs[b][pl.ds(i, 16)]
                t = tbufs[b][pl.ds(i, 16)]
                m = mb[n, pl.ds(i, 16)]
                ad = jnp.abs(p - t)
                adm = ad * m
                cm = jnp.minimum(adm, m)
                acc_sm = acc_sm + cm * (adm - 0.5 * cm)
                acc_ad = acc_ad + adm
            return (acc_sm, acc_ad)

        return lax.fori_loop(0, _HSL // 64, inner, acc)

    def pair(q2, acc):
        acc = chunk(q2 * 2, 0, acc)
        acc = chunk(q2 * 2 + 1, 1, acc)
        return acc

    zero = jnp.zeros((16,), jnp.float32)
    acc_sm, acc_ad = lax.fori_loop(0, nq // 2, pair, (zero, zero))

    accb[0, :] = acc_sm
    accb[1, :] = acc_ad
    pltpu.sync_copy(accb, out_hbm.at[wid])


def _sc_partials(p_flat, t_flat, m2):
    mesh = plsc.VectorSubcoreMesh(core_axis_name="c", subcore_axis_name="s")
    f = pl.kernel(
        _sc_body,
        mesh=mesh,
        out_type=jax.ShapeDtypeStruct((_NW, 2, 16), jnp.float32),
        scratch_types=[
            pltpu.VMEM((_HSL,), jnp.float32),
            pltpu.VMEM((_HSL,), jnp.float32),
            pltpu.VMEM((_HSL,), jnp.float32),
            pltpu.VMEM((_HSL,), jnp.float32),
            pltpu.VMEM((_N, _HSL), jnp.float32),
            pltpu.VMEM((2, 16), jnp.float32),
            pltpu.SemaphoreType.DMA,
            pltpu.SemaphoreType.DMA,
            pltpu.SemaphoreType.DMA,
            pltpu.SemaphoreType.DMA,
        ],
    )
    return f(p_flat, t_flat, m2)


def kernel(pred, target, front_position):
    N, C, H, W = pred.shape
    m_f32 = front_position.reshape(N, _R, 128).astype(jnp.float32)

    p3 = pred.reshape(N * C, _R, 128)
    t3 = target.reshape(N * C, _R, 128)
    p_flat = pred.reshape(-1)
    t_flat = target.reshape(-1)
    m2 = m_f32.reshape(N, _R * 128)

    sc = _sc_partials(p_flat, t_flat, m2)
    tc = _tc_partials(p3, t3, m_f32)

    sm = tc[0] + jnp.sum(sc[:, 0, :])
    ad = tc[1] + jnp.sum(sc[:, 1, :])
    cnt = tc[2] * C
    return (sm / cnt * _LOSS_WEIGHT, ad / cnt)


# speedup vs baseline: 5.7561x; 5.7561x over previous
import functools
import jax
import jax.numpy as jnp
from jax.experimental import pallas as pl
from jax.experimental.pallas import tpu as pltpu

_LOSS_WEIGHT = 1.0


def _body(p_ref, t_ref, m_ref, out_ref, *, steps_per_n, C):
    i = pl.program_id(0)
    ni = pl.num_programs(0)

    @pl.when(i == 0)
    def _init():
        out_ref[0] = 0.0
        out_ref[1] = 0.0
        out_ref[2] = 0.0

    m = m_ref[0]
    p = p_ref[...]
    t = t_ref[...]
    ad = jnp.abs(p - t)
    mb = m[None, :, :]
    adm = ad * mb
    cm = jnp.minimum(adm, mb)
    sm = cm * (adm - 0.5 * cm)
    out_ref[0] += jnp.sum(sm)
    out_ref[1] += jnp.sum(adm)

    @pl.when(i % steps_per_n == 0)
    def _cnt():
        out_ref[2] += jnp.sum(m)

    @pl.when(i == ni - 1)
    def _fini():
        cnt = out_ref[2] * C
        out_ref[0] = out_ref[0] / cnt * _LOSS_WEIGHT
        out_ref[1] = out_ref[1] / cnt


def kernel(pred, target, front_position):
    N, C, H, W = pred.shape
    PB = 4
    steps_per_n = C // PB
    nsteps = N * steps_per_n

    p3 = pred.reshape(N * C, H, W)
    t3 = target.reshape(N * C, H, W)
    m3 = front_position.reshape(N, H, W).astype(jnp.float32)

    out = pl.pallas_call(
        functools.partial(_body, steps_per_n=steps_per_n, C=C),
        grid=(nsteps,),
        in_specs=[
            pl.BlockSpec((PB, H, W), lambda i: (i, 0, 0)),
            pl.BlockSpec((PB, H, W), lambda i: (i, 0, 0)),
            pl.BlockSpec((1, H, W), lambda i: (i // steps_per_n, 0, 0)),
        ],
        out_specs=pl.BlockSpec(memory_space=pltpu.SMEM),
        out_shape=jax.ShapeDtypeStruct((3,), jnp.float32),
    )(p3, t3, m3)

    return (out[0], out[1])


# speedup vs baseline: 5.9207x; 1.0286x over previous
import functools
import jax
import jax.numpy as jnp
from jax import lax
from jax.experimental import pallas as pl
from jax.experimental.pallas import tpu as pltpu

_LOSS_WEIGHT = 1.0
_S = 16


def _body(p_ref, t_ref, m_ref, out_ref, *, PB, steps_per_n, C, H, W):
    i = pl.program_id(0)
    ni = pl.num_programs(0)

    @pl.when(i == 0)
    def _init():
        out_ref[0] = 0.0
        out_ref[1] = 0.0
        out_ref[2] = 0.0

    def strip(s, carry):
        acc_sm, acc_ad = carry
        r0 = s * _S
        m = m_ref[0, pl.ds(r0, _S), :]
        for c in range(PB):
            p = p_ref[c, pl.ds(r0, _S), :]
            t = t_ref[c, pl.ds(r0, _S), :]
            ad = jnp.abs(p - t)
            adm = ad * m
            cmn = jnp.minimum(adm, m)
            acc_sm = acc_sm + cmn * (adm - 0.5 * cmn)
            acc_ad = acc_ad + adm
        return (acc_sm, acc_ad)

    z = jnp.zeros((_S, W), jnp.float32)
    acc_sm, acc_ad = lax.fori_loop(0, H // _S, strip, (z, z))
    out_ref[0] += jnp.sum(acc_sm)
    out_ref[1] += jnp.sum(acc_ad)

    @pl.when(i % steps_per_n == 0)
    def _cnt():
        out_ref[2] += jnp.sum(m_ref[0])

    @pl.when(i == ni - 1)
    def _fini():
        cnt = out_ref[2] * C
        out_ref[0] = out_ref[0] / cnt * _LOSS_WEIGHT
        out_ref[1] = out_ref[1] / cnt


def kernel(pred, target, front_position):
    N, C, H, W = pred.shape
    PB = 4
    steps_per_n = C // PB
    nsteps = N * steps_per_n

    p3 = pred.reshape(N * C, H, W)
    t3 = target.reshape(N * C, H, W)
    m3 = front_position.reshape(N, H, W).astype(jnp.float32)

    out = pl.pallas_call(
        functools.partial(_body, PB=PB, steps_per_n=steps_per_n, C=C, H=H, W=W),
        grid=(nsteps,),
        in_specs=[
            pl.BlockSpec((PB, H, W), lambda i: (i, 0, 0)),
            pl.BlockSpec((PB, H, W), lambda i: (i, 0, 0)),
            pl.BlockSpec((1, H, W), lambda i: (i // steps_per_n, 0, 0)),
        ],
        out_specs=pl.BlockSpec(memory_space=pltpu.SMEM),
        out_shape=jax.ShapeDtypeStruct((3,), jnp.float32),
    )(p3, t3, m3)

    return (out[0], out[1])


# speedup vs baseline: 7.2409x; 1.2230x over previous
import functools
import jax
import jax.numpy as jnp
from jax import lax
from jax.experimental import pallas as pl
from jax.experimental.pallas import tpu as pltpu

_LOSS_WEIGHT = 1.0
_S = 16


def _body(p_ref, t_ref, m_ref, out_ref, *, PB, steps_per_n, C, H, W):
    i = pl.program_id(0)
    ni = pl.num_programs(0)

    @pl.when(i == 0)
    def _init():
        out_ref[0] = 0.0
        out_ref[1] = 0.0
        out_ref[2] = 0.0

    def strip(s, carry):
        acc_sm, acc_ad = carry
        r0 = s * _S
        m = m_ref[0, pl.ds(r0, _S), :]
        for c in range(PB):
            p = p_ref[c, pl.ds(r0, _S), :]
            t = t_ref[c, pl.ds(r0, _S), :]
            ad = jnp.abs(p - t)
            adm = ad * m
            cmn = jnp.minimum(adm, m)
            acc_sm = acc_sm + cmn * (adm - 0.5 * cmn)
            acc_ad = acc_ad + adm
        return (acc_sm, acc_ad)

    z = jnp.zeros((_S, W), jnp.float32)
    acc_sm, acc_ad = lax.fori_loop(0, H // _S, strip, (z, z))
    out_ref[0] += jnp.sum(acc_sm)
    out_ref[1] += jnp.sum(acc_ad)

    @pl.when(i % steps_per_n == 0)
    def _cnt():
        out_ref[2] += jnp.sum(m_ref[0])

    @pl.when(i == ni - 1)
    def _fini():
        cnt = out_ref[2] * C
        out_ref[0] = out_ref[0] / cnt * _LOSS_WEIGHT
        out_ref[1] = out_ref[1] / cnt


def kernel(pred, target, front_position):
    N, C, H, W = pred.shape
    PB = 8
    steps_per_n = C // PB
    nsteps = N * steps_per_n

    p3 = pred.reshape(N * C, H, W)
    t3 = target.reshape(N * C, H, W)
    m3 = front_position.reshape(N, H, W).astype(jnp.float32)

    out = pl.pallas_call(
        functools.partial(_body, PB=PB, steps_per_n=steps_per_n, C=C, H=H, W=W),
        grid=(nsteps,),
        in_specs=[
            pl.BlockSpec((PB, H, W), lambda i: (i, 0, 0)),
            pl.BlockSpec((PB, H, W), lambda i: (i, 0, 0)),
            pl.BlockSpec((1, H, W), lambda i: (i // steps_per_n, 0, 0)),
        ],
        out_specs=pl.BlockSpec(memory_space=pltpu.SMEM),
        out_shape=jax.ShapeDtypeStruct((3,), jnp.float32),
    )(p3, t3, m3)

    return (out[0], out[1])
